# Initial kernel scaffold; baseline (speedup 1.0000x reference)
import jax
import jax.numpy as jnp
from jax.experimental import pallas as pl


def kernel(value, value_spatial_shapes, sampling_locations, attention_weights):
    raise NotImplementedError("write your pallas kernel here")


# speedup vs baseline: 81.5271x; 81.5271x over previous
import functools

import jax
import jax.numpy as jnp
from jax import lax
from jax.experimental import pallas as pl
from jax.experimental.pallas import tpu as pltpu
from jax.experimental.pallas import tpu_sc as plsc

BS, NH, HD, NQ, NL, NP = 2, 8, 32, 5440, 4, 4
NK = 5440
QC = 544
NCHUNK = NQ // QC
NW = 32


def _sc_body(vt_hbm, gx_hbm, gy_hbm, aw_hbm, out_hbm,
             vtab, gxv, gyv, awv, outv):
    wid = lax.axis_index("s") * 2 + lax.axis_index("c")
    pair = wid // 2

    pltpu.sync_copy(vt_hbm.at[wid], vtab)

    lane = lax.iota(jnp.int32, 16)
    level = lane >> 2
    wi = jnp.where(level == 0, 64,
         jnp.where(level == 1, 32,
         jnp.where(level == 2, 16, 8)))
    wf = wi.astype(jnp.float32)
    wm1 = wi - 1
    base = jnp.where(level == 0, 0,
           jnp.where(level == 1, 4096,
           jnp.where(level == 2, 5120, 5376)))

    def chunk_body(ci, carry):
        q0 = ci * QC
        pltpu.sync_copy(gx_hbm.at[pair, pl.ds(q0, QC)], gxv)
        pltpu.sync_copy(gy_hbm.at[pair, pl.ds(q0, QC)], gyv)
        pltpu.sync_copy(aw_hbm.at[pair, pl.ds(q0, QC)], awv)

        def q_body(q, c2):
            px = gxv[q] * wf - 0.5
            py = gyv[q] * wf - 0.5
            a = awv[q]
            x0 = px.astype(jnp.int32)
            x0 = jnp.where(x0.astype(jnp.float32) > px, x0 - 1, x0)
            y0 = py.astype(jnp.int32)
            y0 = jnp.where(y0.astype(jnp.float32) > py, y0 - 1, y0)
            fx = px - x0.astype(jnp.float32)
            fy = py - y0.astype(jnp.float32)
            ex = 1.0 - fx
            ey = 1.0 - fy
            x1 = x0 + 1
            y1 = y0 + 1
            vx0 = (x0 >= 0) & (x0 < wi)
            vx1 = (x1 >= 0) & (x1 < wi)
            vy0 = (y0 >= 0) & (y0 < wi)
            vy1 = (y1 >= 0) & (y1 < wi)
            xc0 = jnp.clip(x0, 0, wm1)
            xc1 = jnp.clip(x1, 0, wm1)
            ry0 = base + jnp.clip(y0, 0, wm1) * wi
            ry1 = base + jnp.clip(y1, 0, wm1) * wi
            zero = jnp.zeros((16,), jnp.float32)
            corners = [
                (ry0 + xc0, jnp.where(vx0 & vy0, ex * ey * a, zero)),
                (ry0 + xc1, jnp.where(vx1 & vy0, fx * ey * a, zero)),
                (ry1 + xc0, jnp.where(vx0 & vy1, ex * fy * a, zero)),
                (ry1 + xc1, jnp.where(vx1 & vy1, fx * fy * a, zero)),
            ]
            accs = [jnp.zeros((16,), jnp.float32) for _ in range(8)]
            j = 0
            for rvec, wvec in corners:
                for k in range(16):
                    accs[j % 8] = accs[j % 8] + vtab[rvec[k]] * wvec[k]
                    j += 1
            acc = ((accs[0] + accs[1]) + (accs[2] + accs[3])) + \
                  ((accs[4] + accs[5]) + (accs[6] + accs[7]))
            outv[q] = acc
            return c2

        lax.fori_loop(0, QC, q_body, 0)
        pltpu.sync_copy(outv, out_hbm.at[wid, pl.ds(q0, QC)])
        return carry

    lax.fori_loop(0, NCHUNK, chunk_body, 0)


@jax.jit
def _msda(vt, gx, gy, aw):
    mesh = plsc.VectorSubcoreMesh(core_axis_name="c", subcore_axis_name="s")
    run = functools.partial(
        pl.kernel,
        out_type=jax.ShapeDtypeStruct((NW, NQ, 16), jnp.float32),
        mesh=mesh,
        scratch_types=[
            pltpu.VMEM((NK, 16), jnp.float32),
            pltpu.VMEM((QC, 16), jnp.float32),
            pltpu.VMEM((QC, 16), jnp.float32),
            pltpu.VMEM((QC, 16), jnp.float32),
            pltpu.VMEM((QC, 16), jnp.float32),
        ],
        compiler_params=pltpu.CompilerParams(use_tc_tiling_on_sc=False),
    )(_sc_body)
    return run(vt, gx, gy, aw)


def kernel(value, value_spatial_shapes, sampling_locations, attention_weights):
    vt = (value.transpose(0, 2, 1, 3)
              .reshape(BS, NH, NK, 2, 16)
              .transpose(0, 1, 3, 2, 4)
              .reshape(NW, NK, 16))
    g = sampling_locations.transpose(0, 2, 1, 3, 4, 5)
    gx = g[..., 0].reshape(BS * NH, NQ, NL * NP)
    gy = g[..., 1].reshape(BS * NH, NQ, NL * NP)
    aw = attention_weights.transpose(0, 2, 1, 3, 4).reshape(BS * NH, NQ, NL * NP)
    out_t = _msda(vt, gx, gy, aw)
    out = (out_t.reshape(BS, NH, 2, NQ, 16)
                .transpose(0, 3, 1, 2, 4)
                .reshape(BS, NQ, NH * HD))
    return out.astype(value.dtype)
